# baseline (speedup 1.0000x reference)
import functools

import jax
import jax.numpy as jnp
from jax import lax
from jax.experimental import pallas as pl
from jax.experimental.pallas import tpu as pltpu
from jax.experimental.pallas import tpu_sc as plsc

_N = 10000
_E = 320000
_D = 128
_H = 64
_G = 64
_OUT = 10

_NC = 2
_NS = 16
_NW = _NC * _NS
_CH = 128
_K = 2
_NB = 2

_EP = 327680
_EPW = _EP // _NW
_NCHUNK = _EPW // _CH
_NP = 10112
_RPT = _NP // _NS
_RZ = _RPT // 8

_FASTC = 0
_CF = 160
_CS = 0
_NCPAD = 16 * (_CF + _CS)
_DF = 96
_DS = 64

_sc_mesh = functools.partial(
    plsc.VectorSubcoreMesh, core_axis_name="c", subcore_axis_name="s")
_sc_params = pltpu.CompilerParams(use_tc_tiling_on_sc=False)


def _make_edge_agg(h):
  @functools.partial(
      pl.kernel,
      mesh=_sc_mesh(),
      compiler_params=_sc_params,
      out_type=jax.ShapeDtypeStruct((_NC, _NP, h), jnp.float32),
      scratch_types=[
          pltpu.VMEM((_CF, _CH), jnp.int32),
          pltpu.VMEM((_CF, _CH), jnp.int32),
          pltpu.VMEM((_NB, _K, _CH, h), jnp.float32),
          pltpu.VMEM((_RZ, h), jnp.float32),
          pltpu.VMEM_SHARED((_NP, h), jnp.float32),
          pltpu.SemaphoreType.DMA,
          pltpu.SemaphoreType.DMA,
          pltpu.SemaphoreType.DMA,
          pltpu.SemaphoreType.DMA,
      ],
  )
  def agg(y_hbm, src_hbm, dst_hbm, out_hbm,
          sidx_v, didx_v, rows_v, zero_v, acc_sh, gs0, gs1, ss0, ss1):
    c = lax.axis_index("c")
    s = lax.axis_index("s")
    is_fast = c == _FASTC
    cbase = s * _CF

    def zfill(i, carry):
      for j in range(h // 16):
        zero_v[i, pl.ds(16 * j, 16)] = jnp.zeros((16,), jnp.float32)
      return carry

    lax.fori_loop(0, _RZ, zfill, 0)
    for q in range(8):
      pltpu.async_copy(zero_v, acc_sh.at[pl.ds(s * _RPT + q * _RZ, _RZ)], gs0)
    for q in range(8):
      pltpu.make_async_copy(zero_v, acc_sh.at[pl.ds(0, _RZ)], gs0).wait()
    plsc.subcore_barrier()

    def gather(t, b, gsem):
      for k in range(_K):
        pltpu.async_copy(y_hbm.at[sidx_v.at[t * _K + k]],
                         rows_v.at[b, k], gsem)

    def wait_g(b, gsem):
      for k in range(_K):
        pltpu.make_async_copy(y_hbm.at[pl.ds(0, _CH)],
                              rows_v.at[b, k], gsem).wait()

    def scatter(t, b, ssem):
      for k in range(_K):
        pltpu.async_copy(rows_v.at[b, k],
                         acc_sh.at[didx_v.at[t * _K + k]], ssem, add=True)

    def wait_s(b, ssem):
      for k in range(_K):
        pltpu.make_async_copy(rows_v.at[b, k],
                              acc_sh.at[pl.ds(0, _CH)], ssem).wait()

    nt = _CF // _K

    @pl.when(is_fast)
    def _run_edges():
      pltpu.sync_copy(src_hbm.at[pl.ds(cbase, _CF)], sidx_v)
      pltpu.sync_copy(dst_hbm.at[pl.ds(cbase, _CF)], didx_v)
      gather(0, 0, gs0)
      gather(1, 1, gs1)

      def body(j, carry):
        t0 = 2 * j
        wait_g(0, gs0)
        scatter(t0, 0, ss0)
        wait_s(0, ss0)
        gather(t0 + 2, 0, gs0)
        wait_g(1, gs1)
        scatter(t0 + 1, 1, ss1)
        wait_s(1, ss1)
        gather(t0 + 3, 1, gs1)
        return carry

      lax.fori_loop(0, nt // 2 - 1, body, 0)
      wait_g(0, gs0)
      scatter(nt - 2, 0, ss0)
      wait_s(0, ss0)
      wait_g(1, gs1)
      scatter(nt - 1, 1, ss1)
      wait_s(1, ss1)

    plsc.subcore_barrier()
    pltpu.sync_copy(acc_sh.at[pl.ds(s * _RPT, _RPT)],
                    out_hbm.at[c].at[pl.ds(s * _RPT, _RPT)])

  return agg


_DH = 16
_DK = 8


@functools.partial(
    pl.kernel,
    mesh=_sc_mesh(),
    compiler_params=_sc_params,
    out_type=jax.ShapeDtypeStruct((_NC, _NP, _DH), jnp.float32),
    scratch_types=[
        pltpu.VMEM((_DF, _CH), jnp.int32),
        pltpu.VMEM((_CH, _DH), jnp.float32),
        pltpu.VMEM_SHARED((_NP, _DH), jnp.float32),
        pltpu.SemaphoreType.DMA,
    ],
)
def _agg_deg(dst_hbm, ones_hbm, zero_hbm, out_hbm,
             didx_v, ones_v, acc_sh, ssem):
  c = lax.axis_index("c")
  s = lax.axis_index("s")
  is_fast = c == _FASTC
  nch = jnp.where(is_fast, _DF, _DS)
  cbase = jnp.where(is_fast, s * _DF, 16 * _DF + s * _DS)
  pltpu.sync_copy(dst_hbm.at[pl.ds(cbase, _DF)], didx_v)
  pltpu.sync_copy(ones_hbm, ones_v)
  pltpu.sync_copy(zero_hbm.at[pl.ds(s * _RPT, _RPT)],
                  acc_sh.at[pl.ds(s * _RPT, _RPT)])
  plsc.subcore_barrier()

  def body(j, carry):
    for k in range(_DK):
      pltpu.async_copy(ones_v, acc_sh.at[didx_v.at[j * _DK + k]], ssem,
                       add=True)
    for k in range(_DK):
      pltpu.make_async_copy(ones_v, acc_sh.at[pl.ds(0, _CH)], ssem).wait()
    return carry

  lax.fori_loop(0, nch // _DK, body, 0)
  plsc.subcore_barrier()
  pltpu.sync_copy(acc_sh.at[pl.ds(s * _RPT, _RPT)],
                  out_hbm.at[c].at[pl.ds(s * _RPT, _RPT)])


_agg_h = _make_edge_agg(_H)


def _t1_body(x_ref, w_ref, dp_ref, y_ref):
  deg = dp_ref[0] + dp_ref[1] + 1.0
  dinv = lax.rsqrt(deg)
  xw = jnp.dot(x_ref[...], w_ref[...], preferred_element_type=jnp.float32)
  y_ref[...] = xw * dinv


def _t2_body(p_ref, y_ref, dp_ref, w_ref, b_ref, y2_ref):
  deg = dp_ref[0] + dp_ref[1] + 1.0
  dinv = lax.rsqrt(deg)
  agg = p_ref[0] + p_ref[1] + y_ref[...]
  hid = jnp.maximum(agg * dinv + b_ref[...], 0.0)
  y2_ref[...] = jnp.dot(hid, w_ref[...],
                        preferred_element_type=jnp.float32) * dinv


def _t3_body(q_ref, y_ref, dp_ref, b_ref, batch_ref, f1w_ref, f1b_ref,
             f2w_ref, f2b_ref, out_ref):
  deg = dp_ref[0] + dp_ref[1] + 1.0
  dinv = lax.rsqrt(deg)
  agg = q_ref[0] + q_ref[1] + y_ref[...]
  hid = jnp.maximum(agg * dinv + b_ref[...], 0.0)
  gids = lax.broadcasted_iota(jnp.int32, (_G, 1), 0)
  pt = (batch_ref[...] == gids).astype(jnp.float32)
  sums = jnp.dot(pt, hid, preferred_element_type=jnp.float32)
  counts = jnp.dot(pt, jnp.ones((_N, 1), jnp.float32),
                   preferred_element_type=jnp.float32)
  pooled = sums / jnp.maximum(counts, 1.0)
  mlp = jnp.maximum(
      jnp.dot(pooled, f1w_ref[...], preferred_element_type=jnp.float32)
      + f1b_ref[...], 0.0)
  out_ref[...] = jnp.dot(mlp, f2w_ref[...],
                         preferred_element_type=jnp.float32) + f2b_ref[...]


def kernel(x, edge_index, batch, W1, b1, W2, b2, fc1_w, fc1_b, fc2_w, fc2_b):
  npad = _NCPAD * _CH - _E
  src = jnp.concatenate([edge_index[0], jnp.full((npad,), _N, jnp.int32)])
  dst = jnp.concatenate([edge_index[1], jnp.full((npad,), _N, jnp.int32)])
  src2d = src.reshape(_NCPAD, _CH)
  dst2d = dst.reshape(_NCPAD, _CH)

  ones_ch = jnp.ones((_CH, _DH), jnp.float32)
  zero_d = jnp.zeros((_NP, _DH), jnp.float32)

  degp = _agg_deg(dst2d, ones_ch, zero_d)
  dp = degp[:, :_N, 0:1]

  y1 = pl.pallas_call(
      _t1_body,
      out_shape=jax.ShapeDtypeStruct((_N, _H), jnp.float32),
  )(x, W1, dp)
  y1p = jnp.concatenate([y1, jnp.zeros((_NP - _N, _H), jnp.float32)])

  p1 = _agg_h(y1p, src2d, dst2d)

  y2 = pl.pallas_call(
      _t2_body,
      out_shape=jax.ShapeDtypeStruct((_N, _H), jnp.float32),
  )(p1[:, :_N, :], y1, dp, W2, b1.reshape(1, _H))
  y2p = jnp.concatenate([y2, jnp.zeros((_NP - _N, _H), jnp.float32)])

  p2 = _agg_h(y2p, src2d, dst2d)

  out = pl.pallas_call(
      _t3_body,
      out_shape=jax.ShapeDtypeStruct((_G, _OUT), jnp.float32),
  )(p2[:, :_N, :], y2, dp, b2.reshape(1, _H), batch.reshape(1, _N),
    fc1_w, fc1_b.reshape(1, 128), fc2_w, fc2_b.reshape(1, _OUT))
  return out


# speedup vs baseline: 1.3446x; 1.3446x over previous
import functools

import jax
import jax.numpy as jnp
from jax import lax
from jax.experimental import pallas as pl
from jax.experimental.pallas import tpu as pltpu
from jax.experimental.pallas import tpu_sc as plsc

_N = 10000
_E = 320000
_D = 128
_H = 64
_G = 64
_OUT = 10

_NC = 2
_NS = 16
_NW = _NC * _NS
_CH = 128
_K = 2
_NB = 2

_EP = 327680
_EPW = _EP // _NW
_NCHUNK = _EPW // _CH
_NP = 10112
_RPT = _NP // _NS
_RZ = _RPT // 8

_FASTC = 0
_CF = 148
_CS = 12
_NCPAD = 16 * (_CF + _CS) + (_CF - _CS)
_DF = 96
_DS = 64

_sc_mesh = functools.partial(
    plsc.VectorSubcoreMesh, core_axis_name="c", subcore_axis_name="s")
_sc_params = pltpu.CompilerParams(use_tc_tiling_on_sc=False)


def _make_edge_agg(h):
  @functools.partial(
      pl.kernel,
      mesh=_sc_mesh(),
      compiler_params=_sc_params,
      out_type=jax.ShapeDtypeStruct((_NC, _NP, h), jnp.float32),
      scratch_types=[
          pltpu.VMEM((_CF, _CH), jnp.int32),
          pltpu.VMEM((_CF, _CH), jnp.int32),
          pltpu.VMEM((_NB, _K, _CH, h), jnp.float32),
          pltpu.VMEM((_RZ, h), jnp.float32),
          pltpu.VMEM_SHARED((_NP, h), jnp.float32),
          pltpu.SemaphoreType.DMA,
          pltpu.SemaphoreType.DMA,
          pltpu.SemaphoreType.DMA,
          pltpu.SemaphoreType.DMA,
      ],
  )
  def agg(y_hbm, src_hbm, dst_hbm, out_hbm,
          sidx_v, didx_v, rows_v, zero_v, acc_sh, gs0, gs1, ss0, ss1):
    c = lax.axis_index("c")
    s = lax.axis_index("s")
    is_fast = c == _FASTC
    nch = jnp.where(is_fast, _CF, _CS)
    cbase = jnp.where(is_fast, s * _CF, 16 * _CF + s * _CS)

    def zfill(i, carry):
      for j in range(h // 16):
        zero_v[i, pl.ds(16 * j, 16)] = jnp.zeros((16,), jnp.float32)
      return carry

    lax.fori_loop(0, _RZ, zfill, 0)
    for q in range(8):
      pltpu.async_copy(zero_v, acc_sh.at[pl.ds(s * _RPT + q * _RZ, _RZ)], gs0)
    for q in range(8):
      pltpu.make_async_copy(zero_v, acc_sh.at[pl.ds(0, _RZ)], gs0).wait()
    plsc.subcore_barrier()

    def gather(t, b, gsem):
      for k in range(_K):
        pltpu.async_copy(y_hbm.at[sidx_v.at[t * _K + k]],
                         rows_v.at[b, k], gsem)

    def wait_g(b, gsem):
      for k in range(_K):
        pltpu.make_async_copy(y_hbm.at[pl.ds(0, _CH)],
                              rows_v.at[b, k], gsem).wait()

    def scatter(t, b, ssem):
      for k in range(_K):
        pltpu.async_copy(rows_v.at[b, k],
                         acc_sh.at[didx_v.at[t * _K + k]], ssem, add=True)

    def wait_s(b, ssem):
      for k in range(_K):
        pltpu.make_async_copy(rows_v.at[b, k],
                              acc_sh.at[pl.ds(0, _CH)], ssem).wait()

    nt = nch // _K
    pltpu.sync_copy(src_hbm.at[pl.ds(cbase, _CF)], sidx_v)
    pltpu.sync_copy(dst_hbm.at[pl.ds(cbase, _CF)], didx_v)
    gather(0, 0, gs0)
    gather(1, 1, gs1)

    def body(j, carry):
      t0 = 2 * j
      wait_g(0, gs0)
      scatter(t0, 0, ss0)
      wait_s(0, ss0)
      gather(t0 + 2, 0, gs0)
      wait_g(1, gs1)
      scatter(t0 + 1, 1, ss1)
      wait_s(1, ss1)
      gather(t0 + 3, 1, gs1)
      return carry

    lax.fori_loop(0, nt // 2 - 1, body, 0)
    wait_g(0, gs0)
    scatter(nt - 2, 0, ss0)
    wait_s(0, ss0)
    wait_g(1, gs1)
    scatter(nt - 1, 1, ss1)
    wait_s(1, ss1)

    plsc.subcore_barrier()
    pltpu.sync_copy(acc_sh.at[pl.ds(s * _RPT, _RPT)],
                    out_hbm.at[c].at[pl.ds(s * _RPT, _RPT)])

  return agg


_DH = 16
_DK = 8


@functools.partial(
    pl.kernel,
    mesh=_sc_mesh(),
    compiler_params=_sc_params,
    out_type=jax.ShapeDtypeStruct((_NC, _NP, _DH), jnp.float32),
    scratch_types=[
        pltpu.VMEM((_DF, _CH), jnp.int32),
        pltpu.VMEM((_CH, _DH), jnp.float32),
        pltpu.VMEM_SHARED((_NP, _DH), jnp.float32),
        pltpu.SemaphoreType.DMA,
    ],
)
def _agg_deg(dst_hbm, ones_hbm, zero_hbm, out_hbm,
             didx_v, ones_v, acc_sh, ssem):
  c = lax.axis_index("c")
  s = lax.axis_index("s")
  is_fast = c == _FASTC
  nch = jnp.where(is_fast, _DF, _DS)
  cbase = jnp.where(is_fast, s * _DF, 16 * _DF + s * _DS)
  pltpu.sync_copy(dst_hbm.at[pl.ds(cbase, _DF)], didx_v)
  pltpu.sync_copy(ones_hbm, ones_v)
  pltpu.sync_copy(zero_hbm.at[pl.ds(s * _RPT, _RPT)],
                  acc_sh.at[pl.ds(s * _RPT, _RPT)])
  plsc.subcore_barrier()

  def body(j, carry):
    for k in range(_DK):
      pltpu.async_copy(ones_v, acc_sh.at[didx_v.at[j * _DK + k]], ssem,
                       add=True)
    for k in range(_DK):
      pltpu.make_async_copy(ones_v, acc_sh.at[pl.ds(0, _CH)], ssem).wait()
    return carry

  lax.fori_loop(0, nch // _DK, body, 0)
  plsc.subcore_barrier()
  pltpu.sync_copy(acc_sh.at[pl.ds(s * _RPT, _RPT)],
                  out_hbm.at[c].at[pl.ds(s * _RPT, _RPT)])


_agg_h = _make_edge_agg(_H)


def _dinv_of(dp_ref):
  deg = dp_ref[0, :_N, 0:1] + dp_ref[1, :_N, 0:1] + 1.0
  return lax.rsqrt(deg)


def _t1_body(x_ref, w_ref, dp_ref, y_ref):
  dinv = _dinv_of(dp_ref)
  xw = jnp.dot(x_ref[...], w_ref[...], preferred_element_type=jnp.float32)
  y_ref[pl.ds(0, _N), :] = xw * dinv
  y_ref[pl.ds(_N, _NP - _N), :] = jnp.zeros((_NP - _N, _H), jnp.float32)


def _t2_body(p_ref, y_ref, dp_ref, w_ref, b_ref, y2_ref):
  dinv = _dinv_of(dp_ref)
  agg = p_ref[0, :_N] + p_ref[1, :_N] + y_ref[pl.ds(0, _N), :]
  hid = jnp.maximum(agg * dinv + b_ref[...], 0.0)
  y2_ref[pl.ds(0, _N), :] = jnp.dot(
      hid, w_ref[...], preferred_element_type=jnp.float32) * dinv
  y2_ref[pl.ds(_N, _NP - _N), :] = jnp.zeros((_NP - _N, _H), jnp.float32)


def _t3_body(q_ref, y_ref, dp_ref, b_ref, batch_ref, f1w_ref, f1b_ref,
             f2w_ref, f2b_ref, out_ref):
  dinv = _dinv_of(dp_ref)
  agg = q_ref[0, :_N] + q_ref[1, :_N] + y_ref[pl.ds(0, _N), :]
  hid = jnp.maximum(agg * dinv + b_ref[...], 0.0)
  gids = lax.broadcasted_iota(jnp.int32, (_G, 1), 0)
  pt = (batch_ref[...] == gids).astype(jnp.float32)
  sums = jnp.dot(pt, hid, preferred_element_type=jnp.float32)
  counts = jnp.dot(pt, jnp.ones((_N, 1), jnp.float32),
                   preferred_element_type=jnp.float32)
  pooled = sums / jnp.maximum(counts, 1.0)
  mlp = jnp.maximum(
      jnp.dot(pooled, f1w_ref[...], preferred_element_type=jnp.float32)
      + f1b_ref[...], 0.0)
  out_ref[...] = jnp.dot(mlp, f2w_ref[...],
                         preferred_element_type=jnp.float32) + f2b_ref[...]


def kernel(x, edge_index, batch, W1, b1, W2, b2, fc1_w, fc1_b, fc2_w, fc2_b):
  npad = _NCPAD * _CH - _E
  src = jnp.concatenate([edge_index[0], jnp.full((npad,), _N, jnp.int32)])
  dst = jnp.concatenate([edge_index[1], jnp.full((npad,), _N, jnp.int32)])
  src2d = src.reshape(_NCPAD, _CH)
  dst2d = dst.reshape(_NCPAD, _CH)

  ones_ch = jnp.ones((_CH, _DH), jnp.float32)
  zero_d = jnp.zeros((_NP, _DH), jnp.float32)

  degp = _agg_deg(dst2d, ones_ch, zero_d)

  y1 = pl.pallas_call(
      _t1_body,
      out_shape=jax.ShapeDtypeStruct((_NP, _H), jnp.float32),
  )(x, W1, degp)

  p1 = _agg_h(y1, src2d, dst2d)

  y2 = pl.pallas_call(
      _t2_body,
      out_shape=jax.ShapeDtypeStruct((_NP, _H), jnp.float32),
  )(p1, y1, degp, W2, b1.reshape(1, _H))

  p2 = _agg_h(y2, src2d, dst2d)

  out = pl.pallas_call(
      _t3_body,
      out_shape=jax.ShapeDtypeStruct((_G, _OUT), jnp.float32),
  )(p2, y2, degp, b2.reshape(1, _H), batch.reshape(1, _N),
    fc1_w, fc1_b.reshape(1, 128), fc2_w, fc2_b.reshape(1, _OUT))
  return out


# speedup vs baseline: 1.3832x; 1.0287x over previous
import functools

import jax
import jax.numpy as jnp
from jax import lax
from jax.experimental import pallas as pl
from jax.experimental.pallas import tpu as pltpu
from jax.experimental.pallas import tpu_sc as plsc

_N = 10000
_E = 320000
_D = 128
_H = 64
_G = 64
_OUT = 10

_NC = 2
_NS = 16
_NW = _NC * _NS
_CH = 128
_K = 2
_NB = 2

_EP = 327680
_EPW = _EP // _NW
_NCHUNK = _EPW // _CH
_NP = 10112
_RPT = _NP // _NS
_RZ = _RPT // 8

_FASTC = 0
_CF = 148
_CS = 12
_DF = 96
_DS = 64
_NCPAD = 16 * (_DF + _DS) + (_DF - _DS)

_sc_mesh = functools.partial(
    plsc.VectorSubcoreMesh, core_axis_name="c", subcore_axis_name="s")
_sc_params = pltpu.CompilerParams(use_tc_tiling_on_sc=False)


def _make_edge_agg(h):
  @functools.partial(
      pl.kernel,
      mesh=_sc_mesh(),
      compiler_params=_sc_params,
      out_type=jax.ShapeDtypeStruct((_NC, _NP, h), jnp.float32),
      scratch_types=[
          pltpu.VMEM((_CF, _CH), jnp.int32),
          pltpu.VMEM((_CF, _CH), jnp.int32),
          pltpu.VMEM((_NB, _K, _CH, h), jnp.float32),
          pltpu.VMEM((_RZ, h), jnp.float32),
          pltpu.VMEM_SHARED((_NP, h), jnp.float32),
          pltpu.SemaphoreType.DMA,
          pltpu.SemaphoreType.DMA,
          pltpu.SemaphoreType.DMA,
          pltpu.SemaphoreType.DMA,
      ],
  )
  def agg(y_hbm, src_hbm, dst_hbm, out_hbm,
          sidx_v, didx_v, rows_v, zero_v, acc_sh, gs0, gs1, ss0, ss1):
    c = lax.axis_index("c")
    s = lax.axis_index("s")
    is_fast = c == _FASTC
    nch = jnp.where(is_fast, _CF, _CS)
    cbase = jnp.where(is_fast, s * _CF, 16 * _CF + s * _CS)

    def zfill(i, carry):
      for j in range(h // 16):
        zero_v[i, pl.ds(16 * j, 16)] = jnp.zeros((16,), jnp.float32)
      return carry

    lax.fori_loop(0, _RZ, zfill, 0)
    for q in range(8):
      pltpu.async_copy(zero_v, acc_sh.at[pl.ds(s * _RPT + q * _RZ, _RZ)], gs0)
    for q in range(8):
      pltpu.make_async_copy(zero_v, acc_sh.at[pl.ds(0, _RZ)], gs0).wait()
    plsc.subcore_barrier()

    def gather(t, b, gsem):
      for k in range(_K):
        pltpu.async_copy(y_hbm.at[sidx_v.at[t * _K + k]],
                         rows_v.at[b, k], gsem)

    def wait_g(b, gsem):
      for k in range(_K):
        pltpu.make_async_copy(y_hbm.at[pl.ds(0, _CH)],
                              rows_v.at[b, k], gsem).wait()

    def scatter(t, b, ssem):
      for k in range(_K):
        pltpu.async_copy(rows_v.at[b, k],
                         acc_sh.at[didx_v.at[t * _K + k]], ssem, add=True)

    def wait_s(b, ssem):
      for k in range(_K):
        pltpu.make_async_copy(rows_v.at[b, k],
                              acc_sh.at[pl.ds(0, _CH)], ssem).wait()

    nt = nch // _K

    @pl.when(is_fast)
    def _load_idx_fast():
      pltpu.sync_copy(src_hbm.at[pl.ds(cbase, _CF)], sidx_v)
      pltpu.sync_copy(dst_hbm.at[pl.ds(cbase, _CF)], didx_v)

    @pl.when(jnp.logical_not(is_fast))
    def _load_idx_slow():
      pltpu.sync_copy(src_hbm.at[pl.ds(cbase, _CS)], sidx_v.at[pl.ds(0, _CS)])
      pltpu.sync_copy(dst_hbm.at[pl.ds(cbase, _CS)], didx_v.at[pl.ds(0, _CS)])
    gather(0, 0, gs0)
    gather(1, 1, gs1)

    def body(j, carry):
      t0 = 2 * j
      wait_g(0, gs0)
      scatter(t0, 0, ss0)
      wait_s(0, ss0)
      gather(t0 + 2, 0, gs0)
      wait_g(1, gs1)
      scatter(t0 + 1, 1, ss1)
      wait_s(1, ss1)
      gather(t0 + 3, 1, gs1)
      return carry

    lax.fori_loop(0, nt // 2 - 1, body, 0)
    wait_g(0, gs0)
    scatter(nt - 2, 0, ss0)
    wait_s(0, ss0)
    wait_g(1, gs1)
    scatter(nt - 1, 1, ss1)
    wait_s(1, ss1)

    plsc.subcore_barrier()
    pltpu.sync_copy(acc_sh.at[pl.ds(s * _RPT, _RPT)],
                    out_hbm.at[c].at[pl.ds(s * _RPT, _RPT)])

  return agg


_DH = 16
_DK = 8


@functools.partial(
    pl.kernel,
    mesh=_sc_mesh(),
    compiler_params=_sc_params,
    out_type=jax.ShapeDtypeStruct((_NC, _NP, _DH), jnp.float32),
    scratch_types=[
        pltpu.VMEM((_DF, _CH), jnp.int32),
        pltpu.VMEM((_CH, _DH), jnp.float32),
        pltpu.VMEM_SHARED((_NP, _DH), jnp.float32),
        pltpu.SemaphoreType.DMA,
    ],
)
def _agg_deg(dst_hbm, ones_hbm, zero_hbm, out_hbm,
             didx_v, ones_v, acc_sh, ssem):
  c = lax.axis_index("c")
  s = lax.axis_index("s")
  is_fast = c == _FASTC
  nch = jnp.where(is_fast, _DF, _DS)
  cbase = jnp.where(is_fast, s * _DF, 16 * _DF + s * _DS)
  pltpu.sync_copy(dst_hbm.at[pl.ds(cbase, _DF)], didx_v)
  pltpu.sync_copy(ones_hbm, ones_v)
  pltpu.sync_copy(zero_hbm.at[pl.ds(s * _RPT, _RPT)],
                  acc_sh.at[pl.ds(s * _RPT, _RPT)])
  plsc.subcore_barrier()

  def body(j, carry):
    for k in range(_DK):
      pltpu.async_copy(ones_v, acc_sh.at[didx_v.at[j * _DK + k]], ssem,
                       add=True)
    for k in range(_DK):
      pltpu.make_async_copy(ones_v, acc_sh.at[pl.ds(0, _CH)], ssem).wait()
    return carry

  lax.fori_loop(0, nch // _DK, body, 0)
  plsc.subcore_barrier()
  pltpu.sync_copy(acc_sh.at[pl.ds(s * _RPT, _RPT)],
                  out_hbm.at[c].at[pl.ds(s * _RPT, _RPT)])


_agg_h = _make_edge_agg(_H)


def _dinv_of(dp_ref):
  deg = dp_ref[0, :_N, 0:1] + dp_ref[1, :_N, 0:1] + 1.0
  return lax.rsqrt(deg)


def _t1_body(x_ref, w_ref, dp_ref, y_ref):
  dinv = _dinv_of(dp_ref)
  xw = jnp.dot(x_ref[...], w_ref[...], preferred_element_type=jnp.float32)
  y_ref[pl.ds(0, _N), :] = xw * dinv
  y_ref[pl.ds(_N, _NP - _N), :] = jnp.zeros((_NP - _N, _H), jnp.float32)


def _t2_body(p_ref, y_ref, dp_ref, w_ref, b_ref, y2_ref):
  dinv = _dinv_of(dp_ref)
  agg = p_ref[0, :_N] + p_ref[1, :_N] + y_ref[pl.ds(0, _N), :]
  hid = jnp.maximum(agg * dinv + b_ref[...], 0.0)
  y2_ref[pl.ds(0, _N), :] = jnp.dot(
      hid, w_ref[...], preferred_element_type=jnp.float32) * dinv
  y2_ref[pl.ds(_N, _NP - _N), :] = jnp.zeros((_NP - _N, _H), jnp.float32)


def _t3_body(q_ref, y_ref, dp_ref, b_ref, batch_ref, f1w_ref, f1b_ref,
             f2w_ref, f2b_ref, out_ref):
  dinv = _dinv_of(dp_ref)
  agg = q_ref[0, :_N] + q_ref[1, :_N] + y_ref[pl.ds(0, _N), :]
  hid = jnp.maximum(agg * dinv + b_ref[...], 0.0)
  gids = lax.broadcasted_iota(jnp.int32, (_G, 1), 0)
  pt = (batch_ref[...] == gids).astype(jnp.float32)
  sums = jnp.dot(pt, hid, preferred_element_type=jnp.float32)
  counts = jnp.dot(pt, jnp.ones((_N, 1), jnp.float32),
                   preferred_element_type=jnp.float32)
  pooled = sums / jnp.maximum(counts, 1.0)
  mlp = jnp.maximum(
      jnp.dot(pooled, f1w_ref[...], preferred_element_type=jnp.float32)
      + f1b_ref[...], 0.0)
  out_ref[...] = jnp.dot(mlp, f2w_ref[...],
                         preferred_element_type=jnp.float32) + f2b_ref[...]


def kernel(x, edge_index, batch, W1, b1, W2, b2, fc1_w, fc1_b, fc2_w, fc2_b):
  npad = _NCPAD * _CH - _E
  src = jnp.concatenate([edge_index[0], jnp.full((npad,), _N, jnp.int32)])
  dst = jnp.concatenate([edge_index[1], jnp.full((npad,), _N, jnp.int32)])
  src2d = src.reshape(_NCPAD, _CH)
  dst2d = dst.reshape(_NCPAD, _CH)

  ones_ch = jnp.ones((_CH, _DH), jnp.float32)
  zero_d = jnp.zeros((_NP, _DH), jnp.float32)

  degp = _agg_deg(dst2d, ones_ch, zero_d)

  y1 = pl.pallas_call(
      _t1_body,
      out_shape=jax.ShapeDtypeStruct((_NP, _H), jnp.float32),
  )(x, W1, degp)

  p1 = _agg_h(y1, src2d, dst2d)

  y2 = pl.pallas_call(
      _t2_body,
      out_shape=jax.ShapeDtypeStruct((_NP, _H), jnp.float32),
  )(p1, y1, degp, W2, b1.reshape(1, _H))

  p2 = _agg_h(y2, src2d, dst2d)

  out = pl.pallas_call(
      _t3_body,
      out_shape=jax.ShapeDtypeStruct((_G, _OUT), jnp.float32),
  )(p2, y2, degp, b2.reshape(1, _H), batch.reshape(1, _N),
    fc1_w, fc1_b.reshape(1, 128), fc2_w, fc2_b.reshape(1, _OUT))
  return out
